# baseline (speedup 1.0000x reference)
import functools

import jax
import jax.numpy as jnp
from jax.experimental import pallas as pl
from jax.experimental.pallas import tpu as pltpu

_BB = 128
_NSEM = 8


def _bcast_kernel(pe_ref, out_ref, buf_ref, sem, *, n_blocks):
    buf_ref[...] = jnp.broadcast_to(pe_ref[...][None, :, :], buf_ref.shape)

    def body(i, _):
        pltpu.make_async_copy(
            buf_ref,
            out_ref.at[pl.ds(i * _BB, _BB)],
            sem.at[i % _NSEM],
        ).start()

        @pl.when(i >= _NSEM)
        def _():
            pltpu.make_async_copy(
                buf_ref,
                out_ref.at[pl.ds((i - _NSEM) * _BB, _BB)],
                sem.at[i % _NSEM],
            ).wait()

        return 0

    jax.lax.fori_loop(0, n_blocks, body, 0)

    def drain(i, _):
        j = n_blocks - _NSEM + i
        pltpu.make_async_copy(
            buf_ref,
            out_ref.at[pl.ds(j * _BB, _BB)],
            sem.at[j % _NSEM],
        ).wait()
        return 0

    jax.lax.fori_loop(0, min(_NSEM, n_blocks), drain, 0)


def kernel(x, pe_weight):
    batch = x.shape[0]
    max_len, d_model = pe_weight.shape
    n_blocks = batch // _BB
    return pl.pallas_call(
        functools.partial(_bcast_kernel, n_blocks=n_blocks),
        in_specs=[pl.BlockSpec(memory_space=pltpu.MemorySpace.VMEM)],
        out_specs=pl.BlockSpec(memory_space=pl.ANY),
        out_shape=jax.ShapeDtypeStruct((batch, max_len, d_model), pe_weight.dtype),
        scratch_shapes=[
            pltpu.VMEM((_BB, max_len, d_model), pe_weight.dtype),
            pltpu.SemaphoreType.DMA((_NSEM,)),
        ],
    )(pe_weight)


# speedup vs baseline: 1.6093x; 1.6093x over previous
import functools

import jax
import jax.numpy as jnp
from jax.experimental import pallas as pl
from jax.experimental.pallas import tpu as pltpu

_BB = 128
_NSEM = 8


def _bcast_kernel(pe_ref, out_ref, buf_ref, sem, *, n_blocks):
    buf_ref[...] = jnp.broadcast_to(pe_ref[...], buf_ref.shape)

    def body(i, _):
        pltpu.make_async_copy(
            buf_ref,
            out_ref.at[pl.ds(i * _BB, _BB)],
            sem.at[i % _NSEM],
        ).start()

        @pl.when(i >= _NSEM)
        def _():
            pltpu.make_async_copy(
                buf_ref,
                out_ref.at[pl.ds((i - _NSEM) * _BB, _BB)],
                sem.at[i % _NSEM],
            ).wait()

        return 0

    jax.lax.fori_loop(0, n_blocks, body, 0)

    def drain(i, _):
        j = n_blocks - _NSEM + i
        pltpu.make_async_copy(
            buf_ref,
            out_ref.at[pl.ds(j * _BB, _BB)],
            sem.at[j % _NSEM],
        ).wait()
        return 0

    jax.lax.fori_loop(0, min(_NSEM, n_blocks), drain, 0)


def kernel(x, pe_weight):
    batch = x.shape[0]
    max_len, d_model = pe_weight.shape
    flat = max_len * d_model
    n_blocks = batch // _BB
    out2d = pl.pallas_call(
        functools.partial(_bcast_kernel, n_blocks=n_blocks),
        in_specs=[pl.BlockSpec(memory_space=pltpu.MemorySpace.VMEM)],
        out_specs=pl.BlockSpec(memory_space=pl.ANY),
        out_shape=jax.ShapeDtypeStruct((batch, flat), pe_weight.dtype),
        scratch_shapes=[
            pltpu.VMEM((_BB, flat), pe_weight.dtype),
            pltpu.SemaphoreType.DMA((_NSEM,)),
        ],
    )(pe_weight.reshape(1, flat))
    return out2d.reshape(batch, max_len, d_model)


# speedup vs baseline: 5.2717x; 3.2757x over previous
import jax
import jax.numpy as jnp
from jax.experimental import pallas as pl


def _bcast_kernel(pe_ref, out_ref):
    out_ref[...] = jnp.broadcast_to(pe_ref[...], out_ref.shape)


def kernel(x, pe_weight):
    batch = x.shape[0]
    max_len, d_model = pe_weight.shape
    flat = max_len * d_model
    rb = 800
    out_t = pl.pallas_call(
        _bcast_kernel,
        grid=(flat // rb,),
        in_specs=[pl.BlockSpec((rb, 1), lambda i: (i, 0))],
        out_specs=pl.BlockSpec((rb, batch), lambda i: (i, 0)),
        out_shape=jax.ShapeDtypeStruct((flat, batch), pe_weight.dtype),
    )(pe_weight.reshape(flat, 1))
    return out_t.T.reshape(batch, max_len, d_model)
